# baseline (speedup 1.0000x reference)
import functools

import jax
import jax.numpy as jnp
from jax.experimental import pallas as pl
from jax.experimental.pallas import tpu as pltpu

NUM_ITEMS = 200
PAD = 0
NEG = -1000000000.0


def _spop_block(item_ref, *refs):
    ban_refs = refs[:-1]
    out_ref = refs[-1]
    B, S, _ = item_ref.shape
    C = NUM_ITEMS

    iota_c = jax.lax.broadcasted_iota(jnp.int32, (B, S, C), 2)
    iota_s = jax.lax.broadcasted_iota(jnp.int32, (B, S, 1), 1)

    item = item_ref[...]
    valid = item != PAD
    posv = jnp.where(valid, iota_s, -1)
    maxpos = jnp.max(posv, axis=1, keepdims=True)
    wt = jnp.where(valid & (posv == maxpos), 0.0, 1.0)
    onehot = jnp.where((item == iota_c) & valid, wt, 0.0)
    counts = jnp.sum(onehot, axis=1, keepdims=True)

    m = jnp.max(counts, axis=2, keepdims=True)
    exprow = jnp.exp(counts - m)

    banned = ban_refs[0][...] == iota_c
    for bref in ban_refs[1:]:
        banned = banned | (bref[...] == iota_c)

    sum_unb = jnp.sum(
        jnp.where(banned, 0.0, exprow), axis=2, keepdims=True
    )
    lse = m + jnp.log(sum_unb)
    out_ref[...] = jnp.where(banned, NEG, counts) - lse


@functools.partial(jax.jit, static_argnames=("interpret",))
def _spop(ban_ids, item_ids, interpret=False):
    N, S, K = ban_ids.shape
    B = 128
    grid = (N // B,)
    item3 = item_ids[:, :, None]
    ban_cols = [ban_ids[:, :, k : k + 1] for k in range(K)]
    spec_s1 = pl.BlockSpec((B, S, 1), lambda i: (i, 0, 0))
    pi = pl.pallas_call(
        _spop_block,
        grid=grid,
        in_specs=[spec_s1] * (1 + K),
        out_specs=pl.BlockSpec((B, S, NUM_ITEMS), lambda i: (i, 0, 0)),
        out_shape=jax.ShapeDtypeStruct((N, S, NUM_ITEMS), jnp.float32),
        compiler_params=pltpu.CompilerParams(
            dimension_semantics=("parallel",),
        ),
        interpret=interpret,
    )(item3, *ban_cols)
    return pi


def kernel(ban_ids, item_ids, aux1, aux2, aux3):
    pi = _spop(ban_ids, item_ids)
    n, s = item_ids.shape
    v = jnp.zeros((n, s, 1), jnp.float32)
    return (pi, v)
